# baseline (speedup 1.0000x reference)
import functools

import jax
import jax.numpy as jnp
from jax import lax
from jax.experimental import pallas as pl
from jax.experimental.pallas import tpu as pltpu
from jax.experimental.pallas import tpu_sc as plsc

_LANES = 16


@functools.lru_cache(maxsize=None)
def _sc_partial_dot(vocab, emb_d, batch):
    info = plsc.get_sparse_core_info()
    nc, ns = info.num_cores, info.num_subcores
    nw = nc * ns
    assert batch % nw == 0
    b_per_w = batch // nw
    assert b_per_w <= 128
    assert emb_d % _LANES == 0
    chunks = emb_d // _LANES

    mesh = plsc.VectorSubcoreMesh(core_axis_name="c", subcore_axis_name="s")

    @functools.partial(
        pl.kernel,
        out_type=jax.ShapeDtypeStruct((nw, _LANES), jnp.float32),
        mesh=mesh,
        scratch_types=[
            pltpu.VMEM((b_per_w,), jnp.int32),
            pltpu.VMEM((b_per_w,), jnp.int32),
            pltpu.VMEM((b_per_w, emb_d), jnp.float32),
            pltpu.VMEM((b_per_w, emb_d), jnp.float32),
            pltpu.VMEM((_LANES,), jnp.float32),
            pltpu.SemaphoreType.DMA,
        ],
        compiler_params=pltpu.CompilerParams(use_tc_tiling_on_sc=False),
    )
    def sc_kernel(focus_hbm, context_hbm, emb_hbm, out_hbm,
                  idx_f, idx_c, rows_f, rows_c, acc_v, sem):
        wid = lax.axis_index("s") * nc + lax.axis_index("c")
        base = wid * b_per_w
        pltpu.sync_copy(focus_hbm.at[pl.ds(base, b_per_w)], idx_f)
        pltpu.sync_copy(context_hbm.at[pl.ds(base, b_per_w)], idx_c)
        cp_f = pltpu.async_copy(emb_hbm.at[idx_f], rows_f, sem)
        cp_c = pltpu.async_copy(emb_hbm.at[idx_c], rows_c, sem)
        cp_f.wait()
        cp_c.wait()

        def body(i, acc):
            for j in range(chunks):
                sl = pl.ds(j * _LANES, _LANES)
                acc = acc + rows_f[i, sl] * rows_c[i, sl]
            return acc

        acc = lax.fori_loop(0, b_per_w, body, jnp.zeros((_LANES,), jnp.float32))
        acc_v[...] = acc
        pltpu.sync_copy(acc_v, out_hbm.at[wid])

    return sc_kernel


def _tc_finish_body(p_ref, o_ref):
    s = jnp.sum(p_ref[...])
    val = jnp.minimum(s, 0.0) - jnp.log(1.0 + jnp.exp(-jnp.abs(s)))
    o_ref[...] = jnp.broadcast_to(val, (1, 1))


_tc_finish = pl.pallas_call(
    _tc_finish_body,
    out_shape=jax.ShapeDtypeStruct((1, 1), jnp.float32),
)


def kernel(focus, context, embeddings):
    focus = focus.astype(jnp.int32)
    context = context.astype(jnp.int32)
    vocab, emb_d = embeddings.shape
    partials = _sc_partial_dot(vocab, emb_d, focus.shape[0])(
        focus, context, embeddings)
    return _tc_finish(partials)
